# Initial kernel scaffold; baseline (speedup 1.0000x reference)
import jax
import jax.numpy as jnp
from jax.experimental import pallas as pl


def kernel(x, edge_index, edge_attr, batch, We, be, W1, b1, W2, b2, gamma, beta, Wr1, br1, Wr2, br2):
    raise NotImplementedError("write your pallas kernel here")


# speedup vs baseline: 1.8445x; 1.8445x over previous
import functools

import jax
import jax.numpy as jnp
from jax import lax
from jax.experimental import pallas as pl
from jax.experimental.pallas import tpu as pltpu
from jax.experimental.pallas import tpu_sc as plsc

N = 10000
E = 320000
D = 128
DE = 16
H = 128
L = 3
G = 64

NC = 2
NS = 16
NW = NC * NS
CW = 128
CPW = 80
EP = NW * CPW * CW
NA = 10240
NPS = NA // NS
NCHK = NPS // CW

_mesh = plsc.VectorSubcoreMesh(core_axis_name="c", subcore_axis_name="s")


def _sc_body(h_hbm, src_hbm, dst_hbm, ep_hbm, out_hbm, srcv, dstv, gv, ev, agg):
    c = lax.axis_index("c")
    s = lax.axis_index("s")
    wid = c * NS + s

    def zrow(r, carry):
        for k in range(8):
            gv[r, pl.ds(k * 16, 16)] = jnp.zeros((16,), jnp.float32)
        return carry

    lax.fori_loop(0, CW, zrow, 0)

    def zchunk(t, carry):
        pltpu.sync_copy(gv, agg.at[pl.ds(s * NPS + t * CW, CW)])
        return carry

    lax.fori_loop(0, NCHK, zchunk, 0)
    plsc.subcore_barrier()

    def chunk(j, carry):
        base = (wid * CPW + j) * CW
        pltpu.sync_copy(src_hbm.at[pl.ds(base, CW)], srcv)
        pltpu.sync_copy(dst_hbm.at[pl.ds(base, CW)], dstv)
        pltpu.sync_copy(h_hbm.at[srcv], gv)
        pltpu.sync_copy(ep_hbm.at[pl.ds(base, CW)], ev)

        def crow(r, carry2):
            for k in range(8):
                sl = pl.ds(k * 16, 16)
                gv[r, sl] = jnp.maximum(gv[r, sl] + ev[r, sl], 0.0)
            return carry2

        lax.fori_loop(0, CW, crow, 0)
        pltpu.sync_copy(gv, agg.at[dstv], add=True)
        return carry

    lax.fori_loop(0, CPW, chunk, 0)
    plsc.subcore_barrier()

    def wchunk(t, carry):
        off = s * NPS + t * CW
        pltpu.sync_copy(agg.at[pl.ds(off, CW)], gv)
        pltpu.sync_copy(gv, out_hbm.at[c, pl.ds(off, CW)])
        return carry

    lax.fori_loop(0, NCHK, wchunk, 0)


_sc_gather_scatter = functools.partial(
    pl.kernel,
    out_type=jax.ShapeDtypeStruct((NC, NA, H), jnp.float32),
    mesh=_mesh,
    scratch_types=[
        pltpu.VMEM((CW,), jnp.int32),
        pltpu.VMEM((CW,), jnp.int32),
        pltpu.VMEM((CW, H), jnp.float32),
        pltpu.VMEM((CW, H), jnp.float32),
        pltpu.VMEM_SHARED((NA, H), jnp.float32),
    ],
)(_sc_body)


_BE = 2048


def _eproj_body(attr_ref, we_ref, be_ref, o0, o1, o2):
    a = attr_ref[...]
    for i, o in enumerate((o0, o1, o2)):
        w = we_ref[pl.ds(i * DE, DE), :]
        o[...] = (jnp.dot(a, w, preferred_element_type=jnp.float32)
                  + be_ref[pl.ds(i, 1), :])


def _eproj(edge_attr_pad, we_cat, be_pad):
    return pl.pallas_call(
        _eproj_body,
        grid=(EP // _BE,),
        in_specs=[
            pl.BlockSpec((_BE, DE), lambda i: (i, 0)),
            pl.BlockSpec((L * DE, H), lambda i: (0, 0)),
            pl.BlockSpec((8, H), lambda i: (0, 0)),
        ],
        out_specs=[pl.BlockSpec((_BE, H), lambda i: (i, 0))] * L,
        out_shape=[jax.ShapeDtypeStruct((EP, H), jnp.float32)] * L,
    )(edge_attr_pad, we_cat, be_pad)


_BN = 2000


def _node_body(h_ref, a0_ref, a1_ref, w1_ref, w2_ref, p_ref, o_ref):
    h = h_ref[...]
    z = h + a0_ref[0] + a1_ref[0]
    t = jnp.maximum(
        jnp.dot(z, w1_ref[...], preferred_element_type=jnp.float32)
        + p_ref[0:1, :], 0.0)
    z2 = (jnp.dot(t, w2_ref[...], preferred_element_type=jnp.float32)
          + p_ref[1:2, :])
    r = jnp.maximum(z2, 0.0) + h
    mu = jnp.mean(r, axis=1, keepdims=True)
    cv = r - mu
    var = jnp.mean(cv * cv, axis=1, keepdims=True)
    o_ref[...] = cv * lax.rsqrt(var + 1e-5) * p_ref[2:3, :] + p_ref[3:4, :]


def _node_update(h, aggpart, w1, w2, pvec):
    return pl.pallas_call(
        _node_body,
        grid=(N // _BN,),
        in_specs=[
            pl.BlockSpec((_BN, H), lambda i: (i, 0)),
            pl.BlockSpec((1, _BN, H), lambda i: (0, i, 0)),
            pl.BlockSpec((1, _BN, H), lambda i: (1, i, 0)),
            pl.BlockSpec((H, H), lambda i: (0, 0)),
            pl.BlockSpec((H, H), lambda i: (0, 0)),
            pl.BlockSpec((8, H), lambda i: (0, 0)),
        ],
        out_specs=pl.BlockSpec((_BN, H), lambda i: (i, 0)),
        out_shape=jax.ShapeDtypeStruct((N, H), jnp.float32),
    )(h, aggpart, aggpart, w1, w2, pvec)


_BP = 2000


def _pool_body(h_ref, b_ref, wr1_ref, wr2_ref, rv_ref, o_ref, acc, cnt):
    pid = pl.program_id(0)

    @pl.when(pid == 0)
    def _init():
        acc[...] = jnp.zeros((G, H), jnp.float32)
        cnt[...] = jnp.zeros((G, H), jnp.float32)

    ids = lax.broadcasted_iota(jnp.int32, (G, _BP), 0)
    b = jnp.reshape(b_ref[...], (1, _BP))
    onehot = (ids == b).astype(jnp.float32)
    acc[...] += jnp.dot(onehot, h_ref[...], preferred_element_type=jnp.float32)
    cnt[...] += jnp.sum(onehot, axis=1, keepdims=True)

    @pl.when(pid == pl.num_programs(0) - 1)
    def _fin():
        pooled = acc[...] / jnp.maximum(cnt[...], 1.0)
        t = jnp.maximum(
            jnp.dot(pooled, wr1_ref[...], preferred_element_type=jnp.float32)
            + rv_ref[0:1, :], 0.0)
        o2 = (jnp.dot(t, wr2_ref[...], preferred_element_type=jnp.float32)
              + rv_ref[1:2, :])
        o_ref[...] = o2[:, 0:1]


def _pool_readout(h, batch3d, wr1, wr2_pad, rvec):
    return pl.pallas_call(
        _pool_body,
        grid=(N // _BP,),
        in_specs=[
            pl.BlockSpec((_BP, H), lambda i: (i, 0)),
            pl.BlockSpec((1, 1, _BP), lambda i: (i, 0, 0)),
            pl.BlockSpec((H, H), lambda i: (0, 0)),
            pl.BlockSpec((H, H), lambda i: (0, 0)),
            pl.BlockSpec((8, H), lambda i: (0, 0)),
        ],
        out_specs=pl.BlockSpec((G, 1), lambda i: (0, 0)),
        out_shape=jax.ShapeDtypeStruct((G, 1), jnp.float32),
        scratch_shapes=[
            pltpu.VMEM((G, H), jnp.float32),
            pltpu.VMEM((G, H), jnp.float32),
        ],
    )(h, batch3d, wr1, wr2_pad, rvec)


def kernel(x, edge_index, edge_attr, batch, We, be, W1, b1, W2, b2,
           gamma, beta, Wr1, br1, Wr2, br2):
    pad = EP - E
    src_pad = jnp.concatenate([edge_index[0], jnp.zeros((pad,), jnp.int32)])
    dst_pad = jnp.concatenate([edge_index[1], jnp.full((pad,), N, jnp.int32)])
    attr_pad = jnp.concatenate(
        [edge_attr, jnp.zeros((pad, DE), jnp.float32)], axis=0)

    we_cat = We.reshape(L * DE, H)
    be_pad = jnp.zeros((8, H), jnp.float32).at[0:L].set(be)
    eps = _eproj(attr_pad, we_cat, be_pad)

    h = x
    for i in range(L):
        aggpart = _sc_gather_scatter(h, src_pad, dst_pad, eps[i])
        pvec = jnp.concatenate(
            [b1[i:i + 1], b2[i:i + 1], gamma[i:i + 1], beta[i:i + 1],
             jnp.zeros((4, H), jnp.float32)], axis=0)
        h = _node_update(h, aggpart, W1[i], W2[i], pvec)

    batch3d = batch.reshape(N // _BP, 1, _BP)
    wr2_pad = jnp.zeros((H, H), jnp.float32).at[:, 0:1].set(Wr2)
    rvec = (jnp.zeros((8, H), jnp.float32)
            .at[0].set(br1).at[1, 0].set(br2[0]))
    return _pool_readout(h, batch3d, Wr1, wr2_pad, rvec)
